# baseline (speedup 1.0000x reference)
import jax
import jax.numpy as jnp
from jax.experimental import pallas as pl

_EPS = 1e-5
_RB = 32
_SCAL = _RB * 128


def _body(a_ref, w_ref, bias_ref, g_ref, beta_ref, o_ref):
    w = w_ref[...]
    bb = bias_ref[...]
    g = g_ref[...]
    beta = beta_ref[...]
    wbar = jnp.mean(w)
    bbar = jnp.mean(bb)
    dw = w - wbar
    db = bb - bbar
    p = jnp.mean(dw * dw)
    q = jnp.mean(dw * db)
    r = jnp.mean(db * db)
    vat = jnp.tile(dw * g, (1, 128))
    vbt = jnp.tile(db * g, (1, 128))
    betat = jnp.tile(beta, (1, 128))
    a = a_ref[0]
    s = jax.lax.rsqrt((a * a) * p + a * (2.0 * q) + (r + _EPS))
    idx = jnp.broadcast_to(
        (jnp.arange(128 * 64, dtype=jnp.int32) // 64)[None, :], (_RB, 128 * 64)
    )
    c1 = jnp.take_along_axis(a * s, idx, axis=1)
    c2 = jnp.take_along_axis(s, idx, axis=1)
    o_ref[0] = c1 * vat + c2 * vbt + betat


def kernel(data, time, weekday, W, b, ln_gamma, ln_beta):
    del time, weekday
    bsz, seq, nodes, _ = data.shape
    size = W.shape[0]
    m = bsz * seq * nodes
    grid = m // _SCAL
    af = data.reshape(grid, _RB, 128)
    row = lambda v: v.reshape(1, size)
    vec_spec = pl.BlockSpec((1, size), lambda i: (0, 0))
    out = pl.pallas_call(
        _body,
        grid=(grid,),
        in_specs=[
            pl.BlockSpec((1, _RB, 128), lambda i: (i, 0, 0)),
            vec_spec, vec_spec, vec_spec, vec_spec,
        ],
        out_specs=pl.BlockSpec((1, _RB, 128 * size), lambda i: (i, 0, 0)),
        out_shape=jax.ShapeDtypeStruct((grid, _RB, 128 * size), jnp.float32),
    )(af, row(W), row(b), row(ln_gamma), row(ln_beta))
    return out.reshape(bsz, seq, nodes, size)
